# baseline (device time: 34804 ns/iter reference)
import jax
import jax.numpy as jnp
from jax import lax
from jax.experimental import pallas as pl
from jax.experimental.pallas import tpu as pltpu

N_CHUNKS = 8
ROWS = 128


def kernel(O, Wo):
    B, S, H, D = O.shape
    K = H * D
    N = Wo.shape[1]
    S_half = S // 2

    def body(o_hbm, wo_ref, out_ref, o_vmem, send_buf, recv_buf,
             copy_sem, send_sems, recv_sems):
        my_x = lax.axis_index("x")
        my_y = lax.axis_index("y")
        my_z = lax.axis_index("z")
        partner = (my_x, my_y, 1 - my_z)

        stage = pltpu.make_async_copy(o_hbm, o_vmem, copy_sem)
        stage.start()

        barrier_sem = pltpu.get_barrier_semaphore()
        pl.semaphore_signal(
            barrier_sem, inc=1,
            device_id=partner, device_id_type=pl.DeviceIdType.MESH,
        )
        pl.semaphore_wait(barrier_sem, 1)

        wo = wo_ref[...].astype(jnp.bfloat16)
        p0 = (1 - my_z) * S_half
        m0 = my_z * S_half
        stage.wait()

        def o_block(b, start, rows):
            blk = o_vmem[b, pl.ds(start, rows), :, :].astype(jnp.bfloat16)
            return jnp.reshape(blk, (rows, K))

        rdmas = []
        for c in range(N_CHUNKS):
            b, r = divmod(c, S_half // ROWS)
            acc = jnp.dot(
                o_block(b, p0 + r * ROWS, ROWS), wo,
                preferred_element_type=jnp.float32,
            )
            send_buf[c] = acc.astype(jnp.bfloat16)
            rdma = pltpu.make_async_remote_copy(
                src_ref=send_buf.at[c],
                dst_ref=recv_buf.at[c],
                send_sem=send_sems.at[c],
                recv_sem=recv_sems.at[c],
                device_id=partner,
                device_id_type=pl.DeviceIdType.MESH,
            )
            rdma.start()
            rdmas.append(rdma)

        for b in range(B):
            out_ref[b] = jnp.dot(
                o_block(b, m0, S_half), wo,
                preferred_element_type=jnp.float32,
            )

        for c in range(N_CHUNKS):
            b, r = divmod(c, S_half // ROWS)
            rdmas[c].wait_recv()
            rows = pl.ds(r * ROWS, ROWS)
            out_ref[b, rows, :] = (
                out_ref[b, rows, :] + recv_buf[c].astype(jnp.float32)
            )
        for c in range(N_CHUNKS):
            rdmas[c].wait_send()

    out_shape = jax.ShapeDtypeStruct((B, S_half, N), jnp.float32)
    return pl.pallas_call(
        body,
        out_shape=out_shape,
        in_specs=[
            pl.BlockSpec(memory_space=pltpu.MemorySpace.HBM),
            pl.BlockSpec(memory_space=pltpu.VMEM),
        ],
        out_specs=pl.BlockSpec(memory_space=pltpu.VMEM),
        scratch_shapes=[
            pltpu.VMEM((B, S, H, D), jnp.float32),
            pltpu.VMEM((N_CHUNKS, ROWS, N), jnp.bfloat16),
            pltpu.VMEM((N_CHUNKS, ROWS, N), jnp.bfloat16),
            pltpu.SemaphoreType.DMA,
            pltpu.SemaphoreType.DMA((N_CHUNKS,)),
            pltpu.SemaphoreType.DMA((N_CHUNKS,)),
        ],
        compiler_params=pltpu.CompilerParams(collective_id=0),
    )(O, Wo)


# device time: 27740 ns/iter; 1.2547x vs baseline; 1.2547x over previous
import jax
import jax.numpy as jnp
from jax import lax
from jax.experimental import pallas as pl
from jax.experimental.pallas import tpu as pltpu

NJ = 4
ROWS = 128


def kernel(O, Wo):
    B, S, H, D = O.shape
    K = H * D
    N = Wo.shape[1]
    S_half = S // 2

    O2 = O.reshape(B, S, K)

    def body(o_ref, wo_ref, out_ref, send_buf, zrecv_buf, precv_buf,
             zsend_sems, zrecv_sems, fsend_sems, frecv_sems):
        my_x = lax.axis_index("x")
        my_y = lax.axis_index("y")
        my_z = lax.axis_index("z")
        z_partner = (my_x, my_y, 1 - my_z)
        x_nbr = (1 - my_x, my_y, my_z)
        y_nbr = (my_x, 1 - my_y, my_z)
        p = (my_x + my_y) % 2

        barrier_sem = pltpu.get_barrier_semaphore()
        for nbr in (z_partner, x_nbr, y_nbr):
            pl.semaphore_signal(
                barrier_sem, inc=1,
                device_id=nbr, device_id_type=pl.DeviceIdType.MESH,
            )
        pl.semaphore_wait(barrier_sem, 3)

        wo = wo_ref[...].astype(jnp.bfloat16)
        p0 = (1 - my_z) * S_half
        m0 = my_z * S_half

        z_rdmas = []
        for j in range(NJ):
            b = 2 * p + j // 2
            r = j % 2
            acc = jnp.dot(
                o_ref[b, pl.ds(p0 + r * ROWS, ROWS), :].astype(jnp.bfloat16),
                wo,
                preferred_element_type=jnp.float32,
            )
            send_buf[j] = acc.astype(jnp.bfloat16)
            rdma = pltpu.make_async_remote_copy(
                src_ref=send_buf.at[j],
                dst_ref=zrecv_buf.at[j],
                send_sem=zsend_sems.at[j],
                recv_sem=zrecv_sems.at[j],
                device_id=z_partner,
                device_id_type=pl.DeviceIdType.MESH,
            )
            rdma.start()
            z_rdmas.append(rdma)

        for b in range(B):
            out_ref[b] = jnp.dot(
                o_ref[b, pl.ds(m0, S_half), :].astype(jnp.bfloat16), wo,
                preferred_element_type=jnp.float32,
            )

        f_rdmas = []
        for j in range(NJ):
            z_rdmas[j].wait_recv()
            fwd = pltpu.make_async_remote_copy(
                src_ref=zrecv_buf.at[j],
                dst_ref=precv_buf.at[j],
                send_sem=fsend_sems.at[j],
                recv_sem=frecv_sems.at[j],
                device_id=x_nbr if j < 2 else y_nbr,
                device_id_type=pl.DeviceIdType.MESH,
            )
            fwd.start()
            f_rdmas.append(fwd)

        for j in range(NJ):
            b = 2 * p + j // 2
            rows = pl.ds((j % 2) * ROWS, ROWS)
            out_ref[b, rows, :] = (
                out_ref[b, rows, :] + zrecv_buf[j].astype(jnp.float32)
            )
        for j in range(NJ):
            b = 2 * (1 - p) + j // 2
            rows = pl.ds((j % 2) * ROWS, ROWS)
            f_rdmas[j].wait_recv()
            out_ref[b, rows, :] = (
                out_ref[b, rows, :] + precv_buf[j].astype(jnp.float32)
            )
        for j in range(NJ):
            z_rdmas[j].wait_send()
            f_rdmas[j].wait_send()

    out_shape = jax.ShapeDtypeStruct((B, S_half, N), jnp.float32)
    return pl.pallas_call(
        body,
        out_shape=out_shape,
        in_specs=[
            pl.BlockSpec(memory_space=pltpu.VMEM),
            pl.BlockSpec(memory_space=pltpu.VMEM),
        ],
        out_specs=pl.BlockSpec(memory_space=pltpu.VMEM),
        scratch_shapes=[
            pltpu.VMEM((NJ, ROWS, N), jnp.bfloat16),
            pltpu.VMEM((NJ, ROWS, N), jnp.bfloat16),
            pltpu.VMEM((NJ, ROWS, N), jnp.bfloat16),
            pltpu.SemaphoreType.DMA((NJ,)),
            pltpu.SemaphoreType.DMA((NJ,)),
            pltpu.SemaphoreType.DMA((NJ,)),
            pltpu.SemaphoreType.DMA((NJ,)),
        ],
        compiler_params=pltpu.CompilerParams(collective_id=0),
    )(O2, Wo)


# device time: 26422 ns/iter; 1.3172x vs baseline; 1.0499x over previous
import jax
import jax.numpy as jnp
from jax import lax
from jax.experimental import pallas as pl
from jax.experimental.pallas import tpu as pltpu

NJ = 8
ROWS = 64


def kernel(O, Wo):
    B, S, H, D = O.shape
    K = H * D
    N = Wo.shape[1]
    S_half = S // 2

    O2 = O.reshape(B, S, K)

    def body(o_ref, wo_ref, out_ref, send_buf, zrecv_buf, precv_buf,
             zsend_sems, zrecv_sems, fsend_sems, frecv_sems):
        my_x = lax.axis_index("x")
        my_y = lax.axis_index("y")
        my_z = lax.axis_index("z")
        z_partner = (my_x, my_y, 1 - my_z)
        x_nbr = (1 - my_x, my_y, my_z)
        y_nbr = (my_x, 1 - my_y, my_z)
        p = (my_x + my_y) % 2

        barrier_sem = pltpu.get_barrier_semaphore()
        for nbr in (z_partner, x_nbr, y_nbr):
            pl.semaphore_signal(
                barrier_sem, inc=1,
                device_id=nbr, device_id_type=pl.DeviceIdType.MESH,
            )
        pl.semaphore_wait(barrier_sem, 3)

        wo = wo_ref[...].astype(jnp.bfloat16)
        p0 = (1 - my_z) * S_half
        m0 = my_z * S_half

        z_rdmas = []
        for j in range(NJ):
            b = 2 * p + j // (NJ // 2)
            r = j % (NJ // 2)
            acc = jnp.dot(
                o_ref[b, pl.ds(p0 + r * ROWS, ROWS), :].astype(jnp.bfloat16),
                wo,
                preferred_element_type=jnp.float32,
            )
            send_buf[j] = acc.astype(jnp.bfloat16)
            rdma = pltpu.make_async_remote_copy(
                src_ref=send_buf.at[j],
                dst_ref=zrecv_buf.at[j],
                send_sem=zsend_sems.at[j],
                recv_sem=zrecv_sems.at[j],
                device_id=z_partner,
                device_id_type=pl.DeviceIdType.MESH,
            )
            rdma.start()
            z_rdmas.append(rdma)

        for b in range(B):
            out_ref[b] = jnp.dot(
                o_ref[b, pl.ds(m0, S_half), :].astype(jnp.bfloat16), wo,
                preferred_element_type=jnp.float32,
            )

        f_rdmas = []
        for j in range(NJ):
            z_rdmas[j].wait_recv()
            fwd = pltpu.make_async_remote_copy(
                src_ref=zrecv_buf.at[j],
                dst_ref=precv_buf.at[j],
                send_sem=fsend_sems.at[j],
                recv_sem=frecv_sems.at[j],
                device_id=x_nbr if j < NJ // 2 else y_nbr,
                device_id_type=pl.DeviceIdType.MESH,
            )
            fwd.start()
            f_rdmas.append(fwd)

        for j in range(NJ):
            b = 2 * p + j // (NJ // 2)
            rows = pl.ds((j % (NJ // 2)) * ROWS, ROWS)
            out_ref[b, rows, :] = (
                out_ref[b, rows, :] + zrecv_buf[j].astype(jnp.float32)
            )
        for j in range(NJ):
            b = 2 * (1 - p) + j // (NJ // 2)
            rows = pl.ds((j % (NJ // 2)) * ROWS, ROWS)
            f_rdmas[j].wait_recv()
            out_ref[b, rows, :] = (
                out_ref[b, rows, :] + precv_buf[j].astype(jnp.float32)
            )
        for j in range(NJ):
            z_rdmas[j].wait_send()
            f_rdmas[j].wait_send()

    out_shape = jax.ShapeDtypeStruct((B, S_half, N), jnp.float32)
    return pl.pallas_call(
        body,
        out_shape=out_shape,
        in_specs=[
            pl.BlockSpec(memory_space=pltpu.VMEM),
            pl.BlockSpec(memory_space=pltpu.VMEM),
        ],
        out_specs=pl.BlockSpec(memory_space=pltpu.VMEM),
        scratch_shapes=[
            pltpu.VMEM((NJ, ROWS, N), jnp.bfloat16),
            pltpu.VMEM((NJ, ROWS, N), jnp.bfloat16),
            pltpu.VMEM((NJ, ROWS, N), jnp.bfloat16),
            pltpu.SemaphoreType.DMA((NJ,)),
            pltpu.SemaphoreType.DMA((NJ,)),
            pltpu.SemaphoreType.DMA((NJ,)),
            pltpu.SemaphoreType.DMA((NJ,)),
        ],
        compiler_params=pltpu.CompilerParams(collective_id=0),
    )(O2, Wo)


# device time: 23708 ns/iter; 1.4680x vs baseline; 1.1145x over previous
import jax
import jax.numpy as jnp
from jax import lax
from jax.experimental import pallas as pl
from jax.experimental.pallas import tpu as pltpu

NK = 4
ROWS = 64


def kernel(O, Wo):
    B, S, H, D = O.shape
    K = H * D
    N = Wo.shape[1]
    S_half = S // 2

    O2 = O.reshape(B, S, K)

    def body(o_ref, wo_ref, out_ref, send_buf, zrecv, xrecv, yrecv,
             zs_sems, zr_sems, fxs_sems, fxr_sems, fys_sems, fyr_sems,
             rxs_sems, rxr_sems, rys_sems, ryr_sems):
        my_x = lax.axis_index("x")
        my_y = lax.axis_index("y")
        my_z = lax.axis_index("z")
        z_partner = (my_x, my_y, 1 - my_z)
        x_nbr = (1 - my_x, my_y, my_z)
        y_nbr = (my_x, 1 - my_y, my_z)
        q = 2 * my_x + my_y
        qx = 2 * (1 - my_x) + my_y
        qy = 2 * my_x + (1 - my_y)
        qd = 2 * (1 - my_x) + (1 - my_y)

        barrier_sem = pltpu.get_barrier_semaphore()
        for nbr in (z_partner, x_nbr, y_nbr):
            pl.semaphore_signal(
                barrier_sem, inc=1,
                device_id=nbr, device_id_type=pl.DeviceIdType.MESH,
            )
        pl.semaphore_wait(barrier_sem, 3)

        wo = wo_ref[...].astype(jnp.bfloat16)
        p0 = (1 - my_z) * S_half
        m0 = my_z * S_half

        mesh = pl.DeviceIdType.MESH
        z_rd = [
            pltpu.make_async_remote_copy(
                src_ref=send_buf.at[k], dst_ref=zrecv.at[k],
                send_sem=zs_sems.at[k], recv_sem=zr_sems.at[k],
                device_id=z_partner, device_id_type=mesh,
            )
            for k in range(NK)
        ]
        fx = [
            pltpu.make_async_remote_copy(
                src_ref=zrecv.at[k], dst_ref=xrecv.at[k],
                send_sem=fxs_sems.at[k], recv_sem=fxr_sems.at[k],
                device_id=x_nbr, device_id_type=mesh,
            )
            for k in range(NK)
        ]
        fy = [
            pltpu.make_async_remote_copy(
                src_ref=zrecv.at[k], dst_ref=yrecv.at[k],
                send_sem=fys_sems.at[k], recv_sem=fyr_sems.at[k],
                device_id=y_nbr, device_id_type=mesh,
            )
            for k in range(NK)
        ]
        rx = [
            pltpu.make_async_remote_copy(
                src_ref=yrecv.at[i], dst_ref=xrecv.at[NK + i],
                send_sem=rxs_sems.at[i], recv_sem=rxr_sems.at[i],
                device_id=x_nbr, device_id_type=mesh,
            )
            for i in range(2)
        ]
        ry = [
            pltpu.make_async_remote_copy(
                src_ref=xrecv.at[2 + i], dst_ref=yrecv.at[NK + i],
                send_sem=rys_sems.at[i], recv_sem=ryr_sems.at[i],
                device_id=y_nbr, device_id_type=mesh,
            )
            for i in range(2)
        ]

        for k in range(NK):
            acc = jnp.dot(
                o_ref[q, pl.ds(p0 + k * ROWS, ROWS), :].astype(jnp.bfloat16),
                wo,
                preferred_element_type=jnp.float32,
            )
            send_buf[k] = acc.astype(jnp.bfloat16)
            z_rd[k].start()

        for k in range(NK):
            out_ref[k] = jnp.dot(
                o_ref[k, pl.ds(m0, S_half), :].astype(jnp.bfloat16), wo,
                preferred_element_type=jnp.float32,
            )
            z_rd[k].wait_recv()
            fx[k].start()
            fy[k].start()

        fy[0].wait_recv()
        rx[0].start()
        fy[1].wait_recv()
        rx[1].start()
        fx[2].wait_recv()
        ry[0].start()
        fx[3].wait_recv()
        ry[1].start()

        for k in range(NK):
            rows = pl.ds(k * ROWS, ROWS)
            out_ref[q, rows, :] = (
                out_ref[q, rows, :] + zrecv[k].astype(jnp.float32)
            )
        fx[0].wait_recv()
        fx[1].wait_recv()
        for k in range(NK):
            rows = pl.ds(k * ROWS, ROWS)
            out_ref[qx, rows, :] = (
                out_ref[qx, rows, :] + xrecv[k].astype(jnp.float32)
            )
        fy[2].wait_recv()
        fy[3].wait_recv()
        for k in range(NK):
            rows = pl.ds(k * ROWS, ROWS)
            out_ref[qy, rows, :] = (
                out_ref[qy, rows, :] + yrecv[k].astype(jnp.float32)
            )
        for i in range(2):
            rx[i].wait_recv()
            rows = pl.ds(i * ROWS, ROWS)
            out_ref[qd, rows, :] = (
                out_ref[qd, rows, :] + xrecv[NK + i].astype(jnp.float32)
            )
        for i in range(2):
            ry[i].wait_recv()
            rows = pl.ds((2 + i) * ROWS, ROWS)
            out_ref[qd, rows, :] = (
                out_ref[qd, rows, :] + yrecv[NK + i].astype(jnp.float32)
            )

        for k in range(NK):
            z_rd[k].wait_send()
            fx[k].wait_send()
            fy[k].wait_send()
        for i in range(2):
            rx[i].wait_send()
            ry[i].wait_send()

    out_shape = jax.ShapeDtypeStruct((B, S_half, N), jnp.float32)
    return pl.pallas_call(
        body,
        out_shape=out_shape,
        in_specs=[
            pl.BlockSpec(memory_space=pltpu.VMEM),
            pl.BlockSpec(memory_space=pltpu.VMEM),
        ],
        out_specs=pl.BlockSpec(memory_space=pltpu.VMEM),
        scratch_shapes=[
            pltpu.VMEM((NK, ROWS, N), jnp.bfloat16),
            pltpu.VMEM((NK, ROWS, N), jnp.bfloat16),
            pltpu.VMEM((NK + 2, ROWS, N), jnp.bfloat16),
            pltpu.VMEM((NK + 2, ROWS, N), jnp.bfloat16),
            pltpu.SemaphoreType.DMA((NK,)),
            pltpu.SemaphoreType.DMA((NK,)),
            pltpu.SemaphoreType.DMA((NK,)),
            pltpu.SemaphoreType.DMA((NK,)),
            pltpu.SemaphoreType.DMA((NK,)),
            pltpu.SemaphoreType.DMA((NK,)),
            pltpu.SemaphoreType.DMA((2,)),
            pltpu.SemaphoreType.DMA((2,)),
            pltpu.SemaphoreType.DMA((2,)),
            pltpu.SemaphoreType.DMA((2,)),
        ],
        compiler_params=pltpu.CompilerParams(collective_id=0),
    )(O2, Wo)


# device time: 23226 ns/iter; 1.4985x vs baseline; 1.0208x over previous
import jax
import jax.numpy as jnp
from jax import lax
from jax.experimental import pallas as pl
from jax.experimental.pallas import tpu as pltpu

NK = 8
NH = NK // 2
ROWS = 32


def kernel(O, Wo):
    B, S, H, D = O.shape
    K = H * D
    N = Wo.shape[1]
    S_half = S // 2

    O2 = O.reshape(B, S, K)

    def body(o_ref, wo_ref, out_ref, send_buf, zrecv, xrecv, yrecv,
             zs_sems, zr_sems, fxs_sems, fxr_sems, fys_sems, fyr_sems,
             rxs_sems, rxr_sems, rys_sems, ryr_sems):
        my_x = lax.axis_index("x")
        my_y = lax.axis_index("y")
        my_z = lax.axis_index("z")
        z_partner = (my_x, my_y, 1 - my_z)
        x_nbr = (1 - my_x, my_y, my_z)
        y_nbr = (my_x, 1 - my_y, my_z)
        q = 2 * my_x + my_y
        qx = 2 * (1 - my_x) + my_y
        qy = 2 * my_x + (1 - my_y)
        qd = 2 * (1 - my_x) + (1 - my_y)

        barrier_sem = pltpu.get_barrier_semaphore()
        for nbr in (z_partner, x_nbr, y_nbr):
            pl.semaphore_signal(
                barrier_sem, inc=1,
                device_id=nbr, device_id_type=pl.DeviceIdType.MESH,
            )
        pl.semaphore_wait(barrier_sem, 3)

        wo = wo_ref[...].astype(jnp.bfloat16)
        p0 = (1 - my_z) * S_half
        m0 = my_z * S_half

        mesh = pl.DeviceIdType.MESH
        z_rd = [
            pltpu.make_async_remote_copy(
                src_ref=send_buf.at[k], dst_ref=zrecv.at[k],
                send_sem=zs_sems.at[k], recv_sem=zr_sems.at[k],
                device_id=z_partner, device_id_type=mesh,
            )
            for k in range(NK)
        ]
        fx = [
            pltpu.make_async_remote_copy(
                src_ref=zrecv.at[k], dst_ref=xrecv.at[k],
                send_sem=fxs_sems.at[k], recv_sem=fxr_sems.at[k],
                device_id=x_nbr, device_id_type=mesh,
            )
            for k in range(NK)
        ]
        fy = [
            pltpu.make_async_remote_copy(
                src_ref=zrecv.at[k], dst_ref=yrecv.at[k],
                send_sem=fys_sems.at[k], recv_sem=fyr_sems.at[k],
                device_id=y_nbr, device_id_type=mesh,
            )
            for k in range(NK)
        ]
        rx = [
            pltpu.make_async_remote_copy(
                src_ref=yrecv.at[i], dst_ref=xrecv.at[NK + i],
                send_sem=rxs_sems.at[i], recv_sem=rxr_sems.at[i],
                device_id=x_nbr, device_id_type=mesh,
            )
            for i in range(NH)
        ]
        ry = [
            pltpu.make_async_remote_copy(
                src_ref=xrecv.at[NH + i], dst_ref=yrecv.at[NK + i],
                send_sem=rys_sems.at[i], recv_sem=ryr_sems.at[i],
                device_id=y_nbr, device_id_type=mesh,
            )
            for i in range(NH)
        ]

        for k in range(NK):
            acc = jnp.dot(
                o_ref[q, pl.ds(p0 + k * ROWS, ROWS), :].astype(jnp.bfloat16),
                wo,
                preferred_element_type=jnp.float32,
            )
            send_buf[k] = acc.astype(jnp.bfloat16)
            z_rd[k].start()

        for k in range(NK):
            if k < B:
                out_ref[k] = jnp.dot(
                    o_ref[k, pl.ds(m0, S_half), :].astype(jnp.bfloat16), wo,
                    preferred_element_type=jnp.float32,
                )
            z_rd[k].wait_recv()
            fx[k].start()
            fy[k].start()

        for i in range(NH):
            fy[i].wait_recv()
            rx[i].start()
        for i in range(NH):
            fx[NH + i].wait_recv()
            ry[i].start()

        for k in range(NK):
            rows = pl.ds(k * ROWS, ROWS)
            out_ref[q, rows, :] = (
                out_ref[q, rows, :] + zrecv[k].astype(jnp.float32)
            )
        for i in range(NH):
            fx[i].wait_recv()
        for k in range(NK):
            rows = pl.ds(k * ROWS, ROWS)
            out_ref[qx, rows, :] = (
                out_ref[qx, rows, :] + xrecv[k].astype(jnp.float32)
            )
        for i in range(NH):
            fy[NH + i].wait_recv()
        for k in range(NK):
            rows = pl.ds(k * ROWS, ROWS)
            out_ref[qy, rows, :] = (
                out_ref[qy, rows, :] + yrecv[k].astype(jnp.float32)
            )
        for i in range(NH):
            rx[i].wait_recv()
            rows = pl.ds(i * ROWS, ROWS)
            out_ref[qd, rows, :] = (
                out_ref[qd, rows, :] + xrecv[NK + i].astype(jnp.float32)
            )
        for i in range(NH):
            ry[i].wait_recv()
            rows = pl.ds((NH + i) * ROWS, ROWS)
            out_ref[qd, rows, :] = (
                out_ref[qd, rows, :] + yrecv[NK + i].astype(jnp.float32)
            )

        for k in range(NK):
            z_rd[k].wait_send()
            fx[k].wait_send()
            fy[k].wait_send()
        for i in range(NH):
            rx[i].wait_send()
            ry[i].wait_send()

    out_shape = jax.ShapeDtypeStruct((B, S_half, N), jnp.float32)
    return pl.pallas_call(
        body,
        out_shape=out_shape,
        in_specs=[
            pl.BlockSpec(memory_space=pltpu.VMEM),
            pl.BlockSpec(memory_space=pltpu.VMEM),
        ],
        out_specs=pl.BlockSpec(memory_space=pltpu.VMEM),
        scratch_shapes=[
            pltpu.VMEM((NK, ROWS, N), jnp.bfloat16),
            pltpu.VMEM((NK, ROWS, N), jnp.bfloat16),
            pltpu.VMEM((NK + NH, ROWS, N), jnp.bfloat16),
            pltpu.VMEM((NK + NH, ROWS, N), jnp.bfloat16),
            pltpu.SemaphoreType.DMA((NK,)),
            pltpu.SemaphoreType.DMA((NK,)),
            pltpu.SemaphoreType.DMA((NK,)),
            pltpu.SemaphoreType.DMA((NK,)),
            pltpu.SemaphoreType.DMA((NK,)),
            pltpu.SemaphoreType.DMA((NK,)),
            pltpu.SemaphoreType.DMA((NH,)),
            pltpu.SemaphoreType.DMA((NH,)),
            pltpu.SemaphoreType.DMA((NH,)),
            pltpu.SemaphoreType.DMA((NH,)),
        ],
        compiler_params=pltpu.CompilerParams(collective_id=0),
    )(O2, Wo)
